# baseline (speedup 1.0000x reference)
import functools
import jax
import jax.numpy as jnp
from jax import lax
from jax.experimental import pallas as pl
from jax.experimental.pallas import tpu as pltpu
from jax.experimental.pallas import tpu_sc as plsc

B, S, EMB = 4096, 200, 64
VOCAB = 1000000
NC, NS = 2, 16
NW = NC * NS
LANES = 16
ECHUNKS = EMB // LANES
BT = B // 128
ST = S // 8
BL = 2
ROWS = BL * 128
NCHUNK = S // BL
NBUF = 2
NT = NCHUNK // NBUF


def _sc_call(xt_flat, wt, pos):
    mesh = plsc.VectorSubcoreMesh(core_axis_name="c", subcore_axis_name="s")

    @functools.partial(
        pl.kernel,
        mesh=mesh,
        compiler_params=pltpu.CompilerParams(
            use_tc_tiling_on_sc=False, needs_layout_passes=False),
        out_type=jax.ShapeDtypeStruct((S, 8, BT, 8, 128), jnp.float32),
        scratch_types=[
            pltpu.VMEM((ROWS,), jnp.int32),
            pltpu.VMEM((ROWS,), jnp.int32),
            pltpu.VMEM((ROWS, EMB), jnp.float32),
            pltpu.VMEM((ROWS, EMB), jnp.float32),
            pltpu.VMEM((BL, 8, 8, 128), jnp.float32),
            pltpu.VMEM((BL, 8, 8, 128), jnp.float32),
            pltpu.VMEM((S, EMB), jnp.float32),
            pltpu.SemaphoreType.DMA,
            pltpu.SemaphoreType.DMA,
            pltpu.SemaphoreType.DMA,
            pltpu.SemaphoreType.DMA,
        ],
    )
    def k(xt_hbm, wt_hbm, pos_hbm, out_hbm,
          idx0, idx1, rows0, rows1, c0, c1, pos_v,
          g0, g1, w0, w1):
        idxs = [idx0, idx1]
        rows = [rows0, rows1]
        cbufs = [c0, c1]
        gs = [g0, g1]
        ws = [w0, w1]
        wid = lax.axis_index("s") * NC + lax.axis_index("c")

        pltpu.sync_copy(pos_hbm, pos_v)

        def chunk_off(j):
            ti = j // 4
            jj = j - 4 * ti
            return ti * (BT * 1024) + wid * 1024 + jj * ROWS

        def chunk_s0(j):
            ti = j // 4
            jj = j - 4 * ti
            return ti * 8 + jj * BL

        def start_gather(j, b):
            pltpu.sync_copy(xt_hbm.at[pl.ds(chunk_off(j), ROWS)], idxs[b])
            pltpu.async_copy(wt_hbm.at[idxs[b]], rows[b], gs[b])

        def wait_gather(b):
            pltpu.make_async_copy(wt_hbm.at[idxs[b]], rows[b], gs[b]).wait()

        def out_slice(j):
            return out_hbm.at[pl.ds(chunk_s0(j), BL), :, wid]

        def start_wb(j, b):
            pltpu.async_copy(cbufs[b], out_slice(j), ws[b])

        def wait_wb(j, b):
            pltpu.make_async_copy(cbufs[b], out_slice(j), ws[b]).wait()

        iota = lax.iota(jnp.int32, LANES)

        def transpose_add(j, b):
            s0 = chunk_s0(j)
            gbuf = rows[b]
            cbuf = cbufs[b]
            for sl in range(BL):
                s = s0 + sl
                i0 = jnp.full((LANES,), sl, jnp.int32)
                for ec in range(ECHUNKS):
                    e0 = ec * LANES
                    i1 = (e0 + iota) // 8
                    i2 = (e0 + iota) % 8
                    lc = e0 + iota
                    p = pos_v[s, pl.ds(e0, LANES)]

                    def body(d, c):
                        diag = (iota + d) & (LANES - 1)
                        for blk in range(8):
                            lr = diag + (sl * 128 + blk * LANES)
                            xg = plsc.load_gather(gbuf, [lr, lc])
                            i3 = diag + blk * LANES
                            plsc.store_scatter(cbuf, [i0, i1, i2, i3], xg + p)
                        return c

                    lax.fori_loop(0, LANES, body, 0, unroll=2)

        for b in range(NBUF):
            start_gather(b, b)

        def body(t, carry):
            for b in range(NBUF):
                j = t * NBUF + b
                wait_gather(b)

                @pl.when(t > 0)
                def _():
                    wait_wb(j - NBUF, b)

                transpose_add(j, b)
                start_wb(j, b)

                @pl.when(t < NT - 1)
                def _():
                    start_gather(j + NBUF, b)
            return carry

        lax.fori_loop(0, NT, body, 0)

        for b in range(NBUF):
            wait_wb((NT - 1) * NBUF + b, b)

    return k(xt_flat, wt, pos)


def kernel(X, word_table, position_embedding):
    xt = X.T.reshape(ST, 8, BT, 128).transpose(0, 2, 1, 3).reshape(-1)
    xt = xt.astype(jnp.int32)
    pos = position_embedding[:S, :]
    o5 = _sc_call(xt, word_table, pos)
    out = o5.transpose(2, 4, 0, 1, 3).reshape(B, S, EMB)
    return out


# speedup vs baseline: 1.0745x; 1.0745x over previous
import functools
import jax
import jax.numpy as jnp
from jax import lax
from jax.experimental import pallas as pl
from jax.experimental.pallas import tpu as pltpu
from jax.experimental.pallas import tpu_sc as plsc

B, S, EMB = 4096, 200, 64
VOCAB = 1000000
NC, NS = 2, 16
NW = NC * NS
LANES = 16
ECHUNKS = EMB // LANES
BT = B // 128
ST = S // 8
BL = 2
ROWS = BL * 128
NCHUNK = S // BL
NBUF = 2
NT = NCHUNK // NBUF


def _sc_call(xt_flat, wt, pos):
    mesh = plsc.VectorSubcoreMesh(core_axis_name="c", subcore_axis_name="s")

    @functools.partial(
        pl.kernel,
        mesh=mesh,
        compiler_params=pltpu.CompilerParams(
            use_tc_tiling_on_sc=False, needs_layout_passes=False),
        out_type=jax.ShapeDtypeStruct((S, 8, BT, 8, 128), jnp.float32),
        scratch_types=[
            pltpu.VMEM((ROWS,), jnp.int32),
            pltpu.VMEM((ROWS,), jnp.int32),
            pltpu.VMEM((ROWS, EMB), jnp.float32),
            pltpu.VMEM((ROWS, EMB), jnp.float32),
            pltpu.VMEM((BL, 8, 8, 128), jnp.float32),
            pltpu.VMEM((BL, 8, 8, 128), jnp.float32),
            pltpu.VMEM((S, EMB), jnp.float32),
            pltpu.SemaphoreType.DMA,
            pltpu.SemaphoreType.DMA,
            pltpu.SemaphoreType.DMA,
            pltpu.SemaphoreType.DMA,
        ],
    )
    def k(xt_hbm, wt_hbm, pos_hbm, out_hbm,
          idx0, idx1, rows0, rows1, c0, c1, pos_v,
          g0, g1, w0, w1):
        idxs = [idx0, idx1]
        rows = [rows0, rows1]
        cbufs = [c0, c1]
        gs = [g0, g1]
        ws = [w0, w1]
        wid = lax.axis_index("s") * NC + lax.axis_index("c")

        pltpu.sync_copy(pos_hbm, pos_v)

        def chunk_off(j):
            ti = j // 4
            jj = j - 4 * ti
            return ti * (BT * 1024) + wid * 1024 + jj * ROWS

        def chunk_s0(j):
            ti = j // 4
            jj = j - 4 * ti
            return ti * 8 + jj * BL

        def start_gather(j, b):
            pltpu.sync_copy(xt_hbm.at[pl.ds(chunk_off(j), ROWS)], idxs[b])
            pltpu.async_copy(wt_hbm.at[idxs[b]], rows[b], gs[b])

        def wait_gather(b):
            pltpu.make_async_copy(wt_hbm.at[idxs[b]], rows[b], gs[b]).wait()

        def out_slice(j):
            return out_hbm.at[pl.ds(chunk_s0(j), BL), :, wid]

        def start_wb(j, b):
            pltpu.async_copy(cbufs[b], out_slice(j), ws[b])

        def wait_wb(j, b):
            pltpu.make_async_copy(cbufs[b], out_slice(j), ws[b]).wait()

        iota = lax.iota(jnp.int32, LANES)

        def transpose_add(j, b):
            s0 = chunk_s0(j)
            gbuf = rows[b]
            cbuf = cbufs[b]
            for sl in range(BL):
                s = s0 + sl
                i0 = jnp.full((LANES,), sl, jnp.int32)
                for ec in range(ECHUNKS):
                    e0 = ec * LANES
                    i1 = (e0 + iota) // 8
                    i2 = (e0 + iota) % 8
                    lc = e0 + iota
                    p = pos_v[s, pl.ds(e0, LANES)]

                    def body(d, c):
                        diag = (iota + d) & (LANES - 1)
                        for blk in range(8):
                            lr = diag + (sl * 128 + blk * LANES)
                            xg = plsc.load_gather(gbuf, [lr, lc])
                            i3 = diag + blk * LANES
                            plsc.store_scatter(cbuf, [i0, i1, i2, i3], xg + p)
                        return c

                    lax.fori_loop(0, LANES, body, 0)

        for b in range(NBUF):
            start_gather(b, b)

        def body(t, carry):
            for b in range(NBUF):
                j = t * NBUF + b
                wait_gather(b)

                @pl.when(t > 0)
                def _():
                    wait_wb(j - NBUF, b)

                transpose_add(j, b)
                start_wb(j, b)

                @pl.when(t < NT - 1)
                def _():
                    start_gather(j + NBUF, b)
            return carry

        lax.fori_loop(0, NT, body, 0)

        for b in range(NBUF):
            wait_wb((NT - 1) * NBUF + b, b)

    return k(xt_flat, wt, pos)


def kernel(X, word_table, position_embedding):
    xt = X.T.reshape(ST, 8, BT, 128).transpose(0, 2, 1, 3).reshape(-1)
    xt = xt.astype(jnp.int32)
    pos = position_embedding[:S, :]
    o5 = _sc_call(xt, word_table, pos)
    out = o5.transpose(2, 4, 0, 1, 3).reshape(B, S, EMB)
    return out
